# baseline (speedup 1.0000x reference)
import functools

import jax
import jax.numpy as jnp
from jax import lax
from jax.experimental import pallas as pl
from jax.experimental.pallas import tpu as pltpu
from jax.experimental.pallas import tpu_sc as plsc

VOCAB = 1000000
EMB = 64
LABELS = 10
BATCH = 16384
SEQ = 200

S0 = 104
S1 = SEQ - S0
NC = 2
NS = 16
NW = NC * NS
B_PER_W = BATCH // NW
RSTAGE = 128
NSTAGE = B_PER_W // RSTAGE
OUT_PAD = 16


def _make_sc_kernel():
    mesh = plsc.VectorSubcoreMesh(core_axis_name="c", subcore_axis_name="s")

    @functools.partial(
        pl.kernel,
        mesh=mesh,
        compiler_params=pltpu.CompilerParams(use_tc_tiling_on_sc=False),
        out_type=jax.ShapeDtypeStruct((BATCH * OUT_PAD,), jnp.float32),
        scratch_types=[
            pltpu.VMEM((RSTAGE, SEQ), jnp.int32),
            pltpu.VMEM((SEQ, EMB), jnp.float32),
            pltpu.VMEM((SEQ, EMB), jnp.float32),
            pltpu.VMEM((LABELS * EMB,), jnp.float32),
            pltpu.VMEM((OUT_PAD,), jnp.float32),
            pltpu.VMEM((B_PER_W * OUT_PAD,), jnp.float32),
            pltpu.SemaphoreType.DMA,
            pltpu.SemaphoreType.DMA,
            pltpu.SemaphoreType.DMA,
            pltpu.SemaphoreType.DMA,
        ],
    )
    def sc_kernel(x_hbm, table_hbm, w_hbm, b_hbm, out_hbm,
                  idx_v, rows_a, rows_b, w_v, b_v, out_v,
                  sa0, sa1, sb0, sb1):
        wid = lax.axis_index("s") * NC + lax.axis_index("c")
        base = wid * B_PER_W

        pltpu.sync_copy(w_hbm, w_v.at[pl.ds(0, LABELS * EMB)])
        pltpu.sync_copy(b_hbm, b_v)
        bias = b_v[...]
        lane = lax.iota(jnp.int32, 16)

        dnums = lax.GatherDimensionNumbers(
            offset_dims=(), collapsed_slice_dims=(0,), start_index_map=(0,))

        def lane_sum(v):
            for sh in (8, 4, 2, 1):
                v = v + lax.gather(v, (lane ^ sh)[:, None], dnums, (1,),
                                   mode=lax.GatherScatterMode.PROMISE_IN_BOUNDS)
            return v

        def fire(r, rows_v, s0, s1):
            cp0 = pltpu.async_copy(table_hbm.at[idx_v.at[r, pl.ds(0, S0)]],
                                   rows_v.at[pl.ds(0, S0)], s0)
            cp1 = pltpu.async_copy(
                table_hbm.at[idx_v.at[r, pl.ds(S0, S1)]],
                rows_v.at[pl.ds(S0, S1)], s1)
            return cp0, cp1

        def process(g, r, rows_v):

            def acc_body(s, accs):
                s8 = s * 8
                out = list(accs)
                for k in range(8):
                    for j in range(4):
                        out[j * 4 + (k & 3)] = (
                            out[j * 4 + (k & 3)]
                            + rows_v[s8 + k, pl.ds(j * 16, 16)])
                return tuple(out)

            zero = jnp.zeros((16,), jnp.float32)
            accs = lax.fori_loop(0, SEQ // 8, acc_body, (zero,) * 16)
            sums = [accs[j * 4] + accs[j * 4 + 1]
                    + (accs[j * 4 + 2] + accs[j * 4 + 3]) for j in range(4)]

            one = jnp.ones((16,), jnp.int32)
            zero_i = jnp.zeros((16,), jnp.int32)
            cntv = zero_i
            for c in range(SEQ // 16):
                chunk = idx_v[r, pl.ds(c * 16, 16)]
                cntv = cntv + jnp.where(chunk != 0, one, zero_i)
            tail = idx_v[r, pl.ds(SEQ - 16, 16)]
            cntv = cntv + jnp.where((tail != 0) & (lane >= 16 - SEQ % 16),
                                    one, zero_i)
            cnt = lane_sum(cntv)
            inv = 1.0 / jnp.maximum(cnt.astype(jnp.float32), 1.0)
            mean = [sums[j] * inv for j in range(4)]

            logits = bias
            for l in range(LABELS):
                p = mean[0] * w_v[pl.ds(l * EMB, 16)]
                for j in range(1, 4):
                    p = p + mean[j] * w_v[pl.ds(l * EMB + j * 16, 16)]
                logits = jnp.where(lane == l, logits + lane_sum(p), logits)
            dst = pl.multiple_of((g * RSTAGE + r) * OUT_PAD, 8)
            out_v[pl.ds(dst, OUT_PAD)] = logits

        for g in range(NSTAGE):
            src = pl.multiple_of(base + g * RSTAGE, 8)
            pltpu.sync_copy(x_hbm.at[pl.ds(src, RSTAGE)], idx_v)

            fire(0, rows_a, sa0, sa1)

            def pair_body(r2, carry):
                ra = r2 * 2
                fire(ra + 1, rows_b, sb0, sb1)
                pltpu.make_async_copy(
                    table_hbm.at[idx_v.at[0, pl.ds(0, S0)]],
                    rows_a.at[pl.ds(0, S0)], sa0).wait()
                pltpu.make_async_copy(
                    table_hbm.at[idx_v.at[0, pl.ds(S0, S1)]],
                    rows_a.at[pl.ds(S0, S1)], sa1).wait()
                process(g, ra, rows_a)

                @pl.when(r2 < RSTAGE // 2 - 1)
                def _():
                    fire(ra + 2, rows_a, sa0, sa1)

                pltpu.make_async_copy(
                    table_hbm.at[idx_v.at[0, pl.ds(0, S0)]],
                    rows_b.at[pl.ds(0, S0)], sb0).wait()
                pltpu.make_async_copy(
                    table_hbm.at[idx_v.at[0, pl.ds(S0, S1)]],
                    rows_b.at[pl.ds(S0, S1)], sb1).wait()
                process(g, ra + 1, rows_b)
                return carry

            lax.fori_loop(0, RSTAGE // 2, pair_body, 0)

        pltpu.sync_copy(out_v,
                        out_hbm.at[pl.ds(base * OUT_PAD, B_PER_W * OUT_PAD)])

    return sc_kernel


_sc_kernel = _make_sc_kernel()


@jax.jit
def kernel(x, table, W, b):
    b_pad = jnp.zeros((OUT_PAD,), jnp.float32).at[:LABELS].set(b)
    out = _sc_kernel(x, table, W.reshape(-1), b_pad)
    return out.reshape(BATCH, OUT_PAD)[:, :LABELS]


# speedup vs baseline: 1.6769x; 1.6769x over previous
import functools

import jax
import jax.numpy as jnp
from jax import lax
from jax.experimental import pallas as pl
from jax.experimental.pallas import tpu as pltpu
from jax.experimental.pallas import tpu_sc as plsc

VOCAB = 1000000
EMB = 64
LABELS = 10
BATCH = 16384
SEQ = 200

S0 = 104
S1 = SEQ - S0
NC = 2
NS = 16
NW = NC * NS
B_PER_W = BATCH // NW
RSTAGE = 128
NSTAGE = B_PER_W // RSTAGE
OUT_PAD = 16


TBLK = 32768
TGRID = -(-VOCAB // TBLK)
VPAD = TGRID * TBLK
HSH = (TBLK // 4).bit_length() - 1


def _tc_transpose(table_t):
    def body(in_ref, out_ref):
        h = TBLK // 4
        for q in range(4):
            t = in_ref[:, q * h:(q + 1) * h].T.astype(jnp.bfloat16)
            u = lax.bitcast_convert_type(t, jnp.uint16)
            lo = u[:, 0:EMB // 2].astype(jnp.uint32)
            hi = u[:, EMB // 2:EMB].astype(jnp.uint32)
            w = lo | (hi << 16)
            out_ref[:, q * (EMB // 2):(q + 1) * (EMB // 2)] = (
                lax.bitcast_convert_type(w, jnp.float32))

    return pl.pallas_call(
        body,
        grid=(TGRID,),
        in_specs=[pl.BlockSpec((EMB, TBLK), lambda i: (0, i))],
        out_specs=pl.BlockSpec((TBLK // 4, 128), lambda i: (i, 0)),
        out_shape=jax.ShapeDtypeStruct((VPAD // 4, 128), jnp.float32),
    )(table_t)


def _make_sc_kernel():
    mesh = plsc.VectorSubcoreMesh(core_axis_name="c", subcore_axis_name="s")

    @functools.partial(
        pl.kernel,
        mesh=mesh,
        compiler_params=pltpu.CompilerParams(use_tc_tiling_on_sc=False),
        out_type=jax.ShapeDtypeStruct((BATCH * OUT_PAD,), jnp.float32),
        scratch_types=[
            pltpu.VMEM((RSTAGE, SEQ), jnp.int32),
            pltpu.VMEM((SEQ, EMB // 2), jnp.float32),
            pltpu.VMEM((SEQ, EMB // 2), jnp.float32),
            pltpu.VMEM((LABELS * EMB,), jnp.float32),
            pltpu.VMEM((OUT_PAD,), jnp.float32),
            pltpu.VMEM((B_PER_W * OUT_PAD,), jnp.float32),
            pltpu.SemaphoreType.DMA,
            pltpu.SemaphoreType.DMA,
            pltpu.SemaphoreType.DMA,
            pltpu.SemaphoreType.DMA,
        ],
    )
    def sc_kernel(x_hbm, table_hbm, w_hbm, b_hbm, out_hbm,
                  idx_v, rows_a, rows_b, w_v, b_v, out_v,
                  sa0, sa1, sb0, sb1):
        wid = lax.axis_index("s") * NC + lax.axis_index("c")
        base = wid * B_PER_W

        pltpu.sync_copy(w_hbm, w_v.at[pl.ds(0, LABELS * EMB)])
        pltpu.sync_copy(b_hbm, b_v)
        bias = b_v[...]
        lane = lax.iota(jnp.int32, 16)

        dnums = lax.GatherDimensionNumbers(
            offset_dims=(), collapsed_slice_dims=(0,), start_index_map=(0,))

        def lane_sum(v):
            for sh in (8, 4, 2, 1):
                v = v + lax.gather(v, (lane ^ sh)[:, None], dnums, (1,),
                                   mode=lax.GatherScatterMode.PROMISE_IN_BOUNDS)
            return v

        def fire(r, rows_v, s0, s1):
            cp0 = pltpu.async_copy(table_hbm.at[idx_v.at[r, pl.ds(0, S0)]],
                                   rows_v.at[pl.ds(0, S0)], s0)
            cp1 = pltpu.async_copy(
                table_hbm.at[idx_v.at[r, pl.ds(S0, S1)]],
                rows_v.at[pl.ds(S0, S1)], s1)
            return cp0, cp1

        def process(g, r, rows_v):

            def acc_body(s, accs):
                s8 = s * 8
                out = list(accs)
                for k in range(8):
                    w0 = lax.bitcast_convert_type(
                        rows_v[s8 + k, pl.ds(0, 16)], jnp.int32)
                    w1 = lax.bitcast_convert_type(
                        rows_v[s8 + k, pl.ds(16, 16)], jnp.int32)
                    d0 = lax.bitcast_convert_type(w0 << 16, jnp.float32)
                    d1 = lax.bitcast_convert_type(w1 << 16, jnp.float32)
                    d2 = lax.bitcast_convert_type(
                        w0 & jnp.int32(-65536), jnp.float32)
                    d3 = lax.bitcast_convert_type(
                        w1 & jnp.int32(-65536), jnp.float32)
                    for j, d in enumerate((d0, d1, d2, d3)):
                        out[j * 4 + (k & 3)] = out[j * 4 + (k & 3)] + d
                return tuple(out)

            zero = jnp.zeros((16,), jnp.float32)
            accs = lax.fori_loop(0, SEQ // 8, acc_body, (zero,) * 16)
            sums = [accs[j * 4] + accs[j * 4 + 1]
                    + (accs[j * 4 + 2] + accs[j * 4 + 3]) for j in range(4)]

            one = jnp.ones((16,), jnp.int32)
            zero_i = jnp.zeros((16,), jnp.int32)
            cntv = zero_i
            for c in range(SEQ // 16):
                chunk = idx_v[r, pl.ds(c * 16, 16)]
                cntv = cntv + jnp.where(chunk != 0, one, zero_i)
            tail = idx_v[r, pl.ds(SEQ - 16, 16)]
            cntv = cntv + jnp.where((tail != 0) & (lane >= 16 - SEQ % 16),
                                    one, zero_i)
            cnt = lane_sum(cntv)
            inv = 1.0 / jnp.maximum(cnt.astype(jnp.float32), 1.0)
            mean = [sums[j] * inv for j in range(4)]

            logits = bias
            for l in range(LABELS):
                p = mean[0] * w_v[pl.ds(l * EMB, 16)]
                for j in range(1, 4):
                    p = p + mean[j] * w_v[pl.ds(l * EMB + j * 16, 16)]
                logits = jnp.where(lane == l, logits + lane_sum(p), logits)
            dst = pl.multiple_of((g * RSTAGE + r) * OUT_PAD, 8)
            out_v[pl.ds(dst, OUT_PAD)] = logits

        for g in range(NSTAGE):
            src = pl.multiple_of(base + g * RSTAGE, 8)
            pltpu.sync_copy(x_hbm.at[pl.ds(src, RSTAGE)], idx_v)

            def remap_body(c, carry):
                col = c * 16
                for rr in range(RSTAGE):
                    v = idx_v[rr, pl.ds(col, 16)]
                    l = ((v & ~(TBLK - 1)) | ((v & (TBLK // 4 - 1)) << 2)
                         | ((v >> HSH) & 3))
                    idx_v[rr, pl.ds(col, 16)] = l
                return carry

            lax.fori_loop(0, SEQ // 16, remap_body, 0)

            def remap_tail(rr, carry):
                v = idx_v[rr, pl.ds(SEQ - 16, 16)]
                l = ((v & ~(TBLK - 1)) | ((v & (TBLK // 4 - 1)) << 2)
                     | ((v >> HSH) & 3))
                idx_v[rr, pl.ds(SEQ - 16, 16)] = jnp.where(
                    lane >= 16 - SEQ % 16, l, v)
                return carry

            lax.fori_loop(0, RSTAGE, remap_tail, 0)

            fire(0, rows_a, sa0, sa1)

            def pair_body(r2, carry):
                ra = r2 * 2
                fire(ra + 1, rows_b, sb0, sb1)
                pltpu.make_async_copy(
                    table_hbm.at[idx_v.at[0, pl.ds(0, S0)]],
                    rows_a.at[pl.ds(0, S0)], sa0).wait()
                pltpu.make_async_copy(
                    table_hbm.at[idx_v.at[0, pl.ds(S0, S1)]],
                    rows_a.at[pl.ds(S0, S1)], sa1).wait()
                process(g, ra, rows_a)

                @pl.when(r2 < RSTAGE // 2 - 1)
                def _():
                    fire(ra + 2, rows_a, sa0, sa1)

                pltpu.make_async_copy(
                    table_hbm.at[idx_v.at[0, pl.ds(0, S0)]],
                    rows_b.at[pl.ds(0, S0)], sb0).wait()
                pltpu.make_async_copy(
                    table_hbm.at[idx_v.at[0, pl.ds(S0, S1)]],
                    rows_b.at[pl.ds(S0, S1)], sb1).wait()
                process(g, ra + 1, rows_b)
                return carry

            lax.fori_loop(0, RSTAGE // 2, pair_body, 0)

        pltpu.sync_copy(out_v,
                        out_hbm.at[pl.ds(base * OUT_PAD, B_PER_W * OUT_PAD)])

    return sc_kernel


_sc_kernel = _make_sc_kernel()


@jax.jit
def kernel(x, table, W, b):
    b_pad = jnp.zeros((OUT_PAD,), jnp.float32).at[:LABELS].set(b)
    table_lin = _tc_transpose(table.T).reshape(VPAD, EMB // 2)
    out = _sc_kernel(x, table_lin, W.reshape(-1), b_pad)
    return out.reshape(BATCH, OUT_PAD)[:, :LABELS]
